# Initial kernel scaffold; baseline (speedup 1.0000x reference)
import jax
import jax.numpy as jnp
from jax.experimental import pallas as pl


def kernel(time, key_times):
    raise NotImplementedError("write your pallas kernel here")


# speedup vs baseline: 1.7781x; 1.7781x over previous
import functools

import jax
import jax.numpy as jnp
from jax import lax
from jax.experimental import pallas as pl
from jax.experimental.pallas import tpu as pltpu
from jax.experimental.pallas import tpu_sc as plsc

NC = 2
NS = 16
NW = NC * NS
L = 16

CHUNK = 8192


@functools.partial(jax.jit, static_argnums=(2, 3))
def _time_indexer_sc(time, key_times, n, k):
    ew = n // NW
    nchunks = ew // CHUNK
    nvec = CHUNK // L

    mesh = plsc.VectorSubcoreMesh(
        core_axis_name="c", subcore_axis_name="s",
        num_cores=NC, num_subcores=NS,
    )

    @functools.partial(
        pl.kernel,
        out_type=(
            jax.ShapeDtypeStruct((n,), jnp.int32),
            jax.ShapeDtypeStruct((n,), jnp.int32),
            jax.ShapeDtypeStruct((n,), jnp.float32),
        ),
        mesh=mesh,
        compiler_params=pltpu.CompilerParams(needs_layout_passes=False),
        scratch_types=[
            pltpu.VMEM((k + L,), jnp.float32),
            pltpu.VMEM((k,), jnp.float32),
            pltpu.VMEM((CHUNK,), jnp.float32),
            pltpu.VMEM((CHUNK,), jnp.int32),
            pltpu.VMEM((CHUNK,), jnp.int32),
            pltpu.VMEM((CHUNK,), jnp.float32),
        ],
    )
    def sc_kernel(time_hbm, key_hbm, lo_hbm, up_hbm, fr_hbm,
                  key_v, rtab_v, in_v, lo_v, up_v, fr_v):
        wid = lax.axis_index("s") * NC + lax.axis_index("c")
        wbase = wid * ew

        pltpu.sync_copy(key_hbm, key_v.at[pl.ds(0, k)])
        maxk = plsc.load_gather(key_v, [jnp.full((L,), k - 1, jnp.int32)])
        key_v[pl.ds(k, L)] = maxk
        for j in range(k // L):
            kk = key_v[pl.ds(j * L, L)]
            kn = key_v[pl.ds(j * L + 1, L)]
            rtab_v[pl.ds(j * L, L)] = 1.0 / (kn - kk)

        def chunk_body(ci, _):
            base = wbase + ci * CHUNK
            pltpu.sync_copy(time_hbm.at[pl.ds(base, CHUNK)], in_v)

            def vec_body(i, _):
                off = i * L
                t = in_v[pl.ds(off, L)]
                mk = key_v[pl.ds(k, L)]
                cpr = jnp.full((L,), -1, jnp.int32)
                s = k // 2
                while s >= 1:
                    probe = cpr + s
                    g = plsc.load_gather(key_v, [probe])
                    cpr = jnp.where(g < t, probe, cpr)
                    s //= 2
                upper = jnp.minimum(cpr + 1, k - 1)
                lower = jnp.where(t < mk, jnp.maximum(cpr, 0),
                                  jnp.full((L,), k - 1, jnp.int32))
                tl = plsc.load_gather(key_v, [lower])
                rt = plsc.load_gather(rtab_v, [lower])
                bd = lower != upper
                fr = jnp.where(bd, (t - tl) * rt, jnp.zeros((L,), jnp.float32))
                lo_v[pl.ds(off, L)] = lower
                up_v[pl.ds(off, L)] = upper
                fr_v[pl.ds(off, L)] = fr
                return 0

            lax.fori_loop(0, nvec, vec_body, 0)
            pltpu.sync_copy(lo_v, lo_hbm.at[pl.ds(base, CHUNK)])
            pltpu.sync_copy(up_v, up_hbm.at[pl.ds(base, CHUNK)])
            pltpu.sync_copy(fr_v, fr_hbm.at[pl.ds(base, CHUNK)])
            return 0

        lax.fori_loop(0, nchunks, chunk_body, 0)

    return sc_kernel(time, key_times)


def kernel(time, key_times):
    n = time.shape[0]
    k = key_times.shape[0]
    return _time_indexer_sc(time, key_times, n, k)


# speedup vs baseline: 6.1608x; 3.4649x over previous
import functools

import jax
import jax.numpy as jnp
from jax import lax
from jax.experimental import pallas as pl
from jax.experimental.pallas import tpu as pltpu
from jax.experimental.pallas import tpu_sc as plsc

NC = 2
NS = 16
NW = NC * NS
L = 16

CHUNK = 8192
NBUF = 2
UNROLL = 8


@functools.partial(jax.jit, static_argnums=(2, 3))
def _time_indexer_sc(time, key_times, n, k):
    ew = n // NW
    nchunks = ew // CHUNK
    ngroups = nchunks // NBUF
    nvec = CHUNK // L

    mesh = plsc.VectorSubcoreMesh(
        core_axis_name="c", subcore_axis_name="s",
        num_cores=NC, num_subcores=NS,
    )

    @functools.partial(
        pl.kernel,
        out_type=(
            jax.ShapeDtypeStruct((n,), jnp.int32),
            jax.ShapeDtypeStruct((n,), jnp.int32),
            jax.ShapeDtypeStruct((n,), jnp.float32),
        ),
        mesh=mesh,
        compiler_params=pltpu.CompilerParams(needs_layout_passes=False),
        scratch_types=[
            pltpu.VMEM((k + L,), jnp.float32),
            pltpu.VMEM((k,), jnp.float32),
            [pltpu.VMEM((CHUNK,), jnp.float32) for _ in range(NBUF)],
            [pltpu.VMEM((CHUNK,), jnp.int32) for _ in range(NBUF)],
            [pltpu.VMEM((CHUNK,), jnp.int32) for _ in range(NBUF)],
            [pltpu.VMEM((CHUNK,), jnp.float32) for _ in range(NBUF)],
            [pltpu.SemaphoreType.DMA for _ in range(NBUF)],
            [pltpu.SemaphoreType.DMA for _ in range(NBUF)],
        ],
    )
    def sc_kernel(time_hbm, key_hbm, lo_hbm, up_hbm, fr_hbm,
                  key_v, rtab_v, in_v, lo_v, up_v, fr_v, in_sem, out_sem):
        wid = lax.axis_index("s") * NC + lax.axis_index("c")
        wbase = wid * ew

        pltpu.sync_copy(key_hbm, key_v.at[pl.ds(0, k)])
        maxk = plsc.load_gather(key_v, [jnp.full((L,), k - 1, jnp.int32)])
        key_v[pl.ds(k, L)] = maxk
        for j in range(k // L):
            kk = key_v[pl.ds(j * L, L)]
            kn = key_v[pl.ds(j * L + 1, L)]
            rtab_v[pl.ds(j * L, L)] = 1.0 / (kn - kk)

        def in_copy(c, b):
            base = wbase + c * CHUNK
            return pltpu.make_async_copy(
                time_hbm.at[pl.ds(base, CHUNK)], in_v[b], in_sem[b])

        def out_copies(c, b):
            base = wbase + c * CHUNK
            return (
                pltpu.make_async_copy(lo_v[b], lo_hbm.at[pl.ds(base, CHUNK)],
                                      out_sem[b]),
                pltpu.make_async_copy(up_v[b], up_hbm.at[pl.ds(base, CHUNK)],
                                      out_sem[b]),
                pltpu.make_async_copy(fr_v[b], fr_hbm.at[pl.ds(base, CHUNK)],
                                      out_sem[b]),
            )

        for b in range(NBUF):
            in_copy(b, b).start()

        def group_body(g, _):
            for b in range(NBUF):
                c = g * NBUF + b
                in_copy(c, b).wait()

                @pl.when(g > 0)
                def _wait_prev_out():
                    for cp in out_copies(c - NBUF, b):
                        cp.wait()

                tin, tlo, tup, tfr = in_v[b], lo_v[b], up_v[b], fr_v[b]

                @plsc.parallel_loop(0, nvec, unroll=UNROLL)
                def _vec(i):
                    off = i * L
                    t = tin[pl.ds(off, L)]
                    cpr = jnp.full((L,), -1, jnp.int32)
                    s = k // 2
                    while s >= 1:
                        probe = cpr + s
                        g_ = plsc.load_gather(key_v, [probe])
                        cpr = jnp.where(g_ < t, probe, cpr)
                        s //= 2
                    upper = jnp.minimum(cpr + 1, k - 1)
                    lower = jnp.where(t < maxk, jnp.maximum(cpr, 0),
                                      jnp.full((L,), k - 1, jnp.int32))
                    tl = plsc.load_gather(key_v, [lower])
                    rt = plsc.load_gather(rtab_v, [lower])
                    fr = jnp.where(lower != upper, (t - tl) * rt,
                                   jnp.zeros((L,), jnp.float32))
                    tlo[pl.ds(off, L)] = lower
                    tup[pl.ds(off, L)] = upper
                    tfr[pl.ds(off, L)] = fr

                for cp in out_copies(c, b):
                    cp.start()

                @pl.when(g < ngroups - 1)
                def _prefetch_next():
                    in_copy(c + NBUF, b).start()
            return 0

        lax.fori_loop(0, ngroups, group_body, 0)

        for b in range(NBUF):
            for cp in out_copies(nchunks - NBUF + b, b):
                cp.wait()

    return sc_kernel(time, key_times)


def kernel(time, key_times):
    n = time.shape[0]
    k = key_times.shape[0]
    return _time_indexer_sc(time, key_times, n, k)


# speedup vs baseline: 9.8794x; 1.6036x over previous
import functools

import jax
import jax.numpy as jnp
from jax import lax
from jax.experimental import pallas as pl
from jax.experimental.pallas import tpu as pltpu
from jax.experimental.pallas import tpu_sc as plsc

NC = 2
NS = 16
NW = NC * NS
L = 16

CHUNK = 8192
NBUF = 2
UNROLL = 16


@functools.partial(jax.jit, static_argnums=(2, 3))
def _time_indexer_sc(time, key_times, n, k):
    ew = n // NW
    nchunks = ew // CHUNK
    ngroups = nchunks // NBUF
    nvec = CHUNK // L

    mesh = plsc.VectorSubcoreMesh(
        core_axis_name="c", subcore_axis_name="s",
        num_cores=NC, num_subcores=NS,
    )

    @functools.partial(
        pl.kernel,
        out_type=(
            jax.ShapeDtypeStruct((n,), jnp.int32),
            jax.ShapeDtypeStruct((n,), jnp.int32),
            jax.ShapeDtypeStruct((n,), jnp.float32),
        ),
        mesh=mesh,
        compiler_params=pltpu.CompilerParams(needs_layout_passes=False),
        scratch_types=[
            pltpu.VMEM((k + L,), jnp.float32),
            pltpu.VMEM((k,), jnp.float32),
            [pltpu.VMEM((CHUNK,), jnp.float32) for _ in range(NBUF)],
            [pltpu.VMEM((CHUNK,), jnp.int32) for _ in range(NBUF)],
            [pltpu.VMEM((CHUNK,), jnp.int32) for _ in range(NBUF)],
            [pltpu.VMEM((CHUNK,), jnp.float32) for _ in range(NBUF)],
            [pltpu.SemaphoreType.DMA for _ in range(NBUF)],
            [pltpu.SemaphoreType.DMA for _ in range(NBUF)],
        ],
    )
    def sc_kernel(time_hbm, key_hbm, lo_hbm, up_hbm, fr_hbm,
                  key_v, rtab_v, in_v, lo_v, up_v, fr_v, in_sem, out_sem):
        wid = lax.axis_index("s") * NC + lax.axis_index("c")
        wbase = wid * ew

        pltpu.sync_copy(key_hbm, key_v.at[pl.ds(0, k)])
        maxk = plsc.load_gather(key_v, [jnp.full((L,), k - 1, jnp.int32)])
        key_v[pl.ds(k, L)] = maxk
        piv1 = plsc.load_gather(key_v, [jnp.full((L,), k // 2 - 1, jnp.int32)])
        piv2a = plsc.load_gather(key_v, [jnp.full((L,), k // 4 - 1, jnp.int32)])
        piv2b = plsc.load_gather(
            key_v, [jnp.full((L,), k // 2 + k // 4 - 1, jnp.int32)])
        for j in range(k // L):
            kk = key_v[pl.ds(j * L, L)]
            kn = key_v[pl.ds(j * L + 1, L)]
            rtab_v[pl.ds(j * L, L)] = 1.0 / (kn - kk)

        def in_copy(c, b):
            base = wbase + c * CHUNK
            return pltpu.make_async_copy(
                time_hbm.at[pl.ds(base, CHUNK)], in_v[b], in_sem[b])

        def out_copies(c, b):
            base = wbase + c * CHUNK
            return (
                pltpu.make_async_copy(lo_v[b], lo_hbm.at[pl.ds(base, CHUNK)],
                                      out_sem[b]),
                pltpu.make_async_copy(up_v[b], up_hbm.at[pl.ds(base, CHUNK)],
                                      out_sem[b]),
                pltpu.make_async_copy(fr_v[b], fr_hbm.at[pl.ds(base, CHUNK)],
                                      out_sem[b]),
            )

        for b in range(NBUF):
            in_copy(b, b).start()

        def group_body(g, _):
            for b in range(NBUF):
                c = g * NBUF + b
                in_copy(c, b).wait()

                @pl.when(g > 0)
                def _wait_prev_out():
                    for cp in out_copies(c - NBUF, b):
                        cp.wait()

                tin, tlo, tup, tfr = in_v[b], lo_v[b], up_v[b], fr_v[b]

                @plsc.parallel_loop(0, nvec, unroll=UNROLL)
                def _vec(i):
                    off = i * L
                    t = tin[pl.ds(off, L)]
                    cpr = jnp.where(piv1 < t,
                                    jnp.full((L,), k // 2 - 1, jnp.int32),
                                    jnp.full((L,), -1, jnp.int32))
                    g2 = jnp.where(piv1 < t, piv2b, piv2a)
                    probe2 = cpr + k // 4
                    cpr = jnp.where(g2 < t, probe2, cpr)
                    s = k // 8
                    while s >= 1:
                        probe = cpr + s
                        g_ = plsc.load_gather(key_v, [probe])
                        cpr = jnp.where(g_ < t, probe, cpr)
                        s //= 2
                    upper = jnp.minimum(cpr + 1, k - 1)
                    lower = jnp.where(t < maxk, jnp.maximum(cpr, 0),
                                      jnp.full((L,), k - 1, jnp.int32))
                    tl = plsc.load_gather(key_v, [lower])
                    rt = plsc.load_gather(rtab_v, [lower])
                    fr = jnp.where(lower != upper, (t - tl) * rt,
                                   jnp.zeros((L,), jnp.float32))
                    tlo[pl.ds(off, L)] = lower
                    tup[pl.ds(off, L)] = upper
                    tfr[pl.ds(off, L)] = fr

                for cp in out_copies(c, b):
                    cp.start()

                @pl.when(g < ngroups - 1)
                def _prefetch_next():
                    in_copy(c + NBUF, b).start()
            return 0

        lax.fori_loop(0, ngroups, group_body, 0)

        for b in range(NBUF):
            for cp in out_copies(nchunks - NBUF + b, b):
                cp.wait()

    return sc_kernel(time, key_times)


def kernel(time, key_times):
    n = time.shape[0]
    k = key_times.shape[0]
    return _time_indexer_sc(time, key_times, n, k)
